# Initial kernel scaffold; baseline (speedup 1.0000x reference)
import jax
import jax.numpy as jnp
from jax.experimental import pallas as pl


def kernel(x, gating_output, gate_up_proj, down_proj):
    raise NotImplementedError("write your pallas kernel here")


# speedup vs baseline: 1.9251x; 1.9251x over previous
import functools

import jax
import jax.numpy as jnp
from jax.experimental import pallas as pl
from jax.experimental.pallas import tpu as pltpu

E = 8
TOPK = 2
T = 256
BT = 64
NB = (TOPK * T) // BT + E - 1
NP = NB * BT


def _routing_kernel(g_ref, be_ref, act_ref, ids_ref, w_ref):
    logits = g_ref[...]
    m = jnp.max(logits, axis=1, keepdims=True)
    p = jnp.exp(logits - m)
    p = p / jnp.sum(p, axis=1, keepdims=True)

    eidx = jax.lax.broadcasted_iota(jnp.int32, (T, E), 1)
    m1 = jnp.max(p, axis=1, keepdims=True)
    a1 = jnp.min(jnp.where(p == m1, eidx, E), axis=1, keepdims=True)
    p2 = jnp.where(eidx == a1, -1.0, p)
    m2 = jnp.max(p2, axis=1, keepdims=True)
    a2 = jnp.min(jnp.where(p2 == m2, eidx, E), axis=1, keepdims=True)
    s = m1 + m2
    w1 = m1 / s
    w2 = m2 / s

    e_pairs = jnp.concatenate([a1, a2], axis=0)
    w_pairs = jnp.concatenate([w1, w2], axis=0)
    tio = jax.lax.broadcasted_iota(jnp.int32, (T, 1), 0).astype(jnp.float32)
    t_pairs = jnp.concatenate([tio, tio], axis=0)

    P2 = TOPK * T
    oh = (e_pairs == jax.lax.broadcasted_iota(jnp.int32, (P2, E), 1))
    ohf = oh.astype(jnp.float32)
    counts = jnp.sum(ohf, axis=0, keepdims=True)
    nblk = jnp.floor((counts + (BT - 1)) * (1.0 / BT))
    er = jax.lax.broadcasted_iota(jnp.int32, (E, E), 0)
    ec = jax.lax.broadcasted_iota(jnp.int32, (E, E), 1)
    ute = (er <= ec).astype(jnp.float32)
    cum = jnp.dot(nblk, ute, preferred_element_type=jnp.float32)
    starts = cum - nblk
    total = jnp.sum(nblk)

    pr = jax.lax.broadcasted_iota(jnp.int32, (P2, P2), 0)
    pc = jax.lax.broadcasted_iota(jnp.int32, (P2, P2), 1)
    lt = (pc <= pr).astype(jnp.float32)
    incl = jnp.dot(lt, ohf, preferred_element_type=jnp.float32)
    rank = jnp.sum((incl - 1.0) * ohf, axis=1, keepdims=True)
    start_slot = jnp.sum(ohf * (starts * BT), axis=1, keepdims=True)
    pos = start_slot + rank

    slot_iota = jax.lax.broadcasted_iota(jnp.int32, (P2, NP), 1).astype(jnp.float32)
    at = (pos == slot_iota).astype(jnp.float32)
    cdims = (((0,), (0,)), ((), ()))
    ids_ref[...] = jax.lax.dot_general(
        at, t_pairs, cdims, preferred_element_type=jnp.float32)
    w_ref[...] = jax.lax.dot_general(
        at, w_pairs, cdims, preferred_element_type=jnp.float32)

    bio = jax.lax.broadcasted_iota(jnp.int32, (1, NB), 1).astype(jnp.float32)
    bclamp = jnp.minimum(bio, total - 1.0)
    cum_col = jnp.transpose(cum)
    owner = jnp.sum((cum_col <= bclamp).astype(jnp.int32), axis=0,
                    keepdims=True)
    be_ref[...] = owner
    act_ref[...] = (bio < total).astype(jnp.int32)


def _moe_kernel(be_ref, act_ref, ids_ref, w_ref, x_ref, gu_ref, dn_ref,
                out_ref, *, ff):
    b = pl.program_id(0)

    @pl.when(b == 0)
    def _init():
        out_ref[...] = jnp.zeros_like(out_ref)

    @pl.when(act_ref[b] > 0)
    def _compute():
        ids = ids_ref[pl.ds(b * BT, BT), :]
        w = w_ref[pl.ds(b * BT, BT), :]
        tcol = jax.lax.broadcasted_iota(jnp.int32, (BT, T), 1).astype(jnp.float32)
        perm = (ids == tcol).astype(jnp.float32)
        xg = jnp.dot(perm, x_ref[...],
                     preferred_element_type=jnp.float32)
        wgu = gu_ref[0]
        cdims = (((1,), (1,)), ((), ()))
        gu = jax.lax.dot_general(xg, wgu, cdims,
                                 preferred_element_type=jnp.float32)
        g = gu[:, :ff]
        u = gu[:, ff:]
        h = g * jax.lax.logistic(g) * u
        dn = dn_ref[0]
        y = jax.lax.dot_general(h, dn, cdims,
                                preferred_element_type=jnp.float32)
        y = y * w
        sdims = (((0,), (0,)), ((), ()))
        out_ref[...] += jax.lax.dot_general(
            perm, y, sdims, preferred_element_type=jnp.float32)


def kernel(x, gating_output, gate_up_proj, down_proj):
    t, d = x.shape
    e = gating_output.shape[1]
    ff2 = gate_up_proj.shape[1]
    ff = ff2 // 2

    be, act, ids, w = pl.pallas_call(
        _routing_kernel,
        out_shape=[
            jax.ShapeDtypeStruct((1, NB), jnp.int32),
            jax.ShapeDtypeStruct((1, NB), jnp.int32),
            jax.ShapeDtypeStruct((NP, 1), jnp.float32),
            jax.ShapeDtypeStruct((NP, 1), jnp.float32),
        ],
    )(gating_output)

    be = be.reshape(NB)
    act = act.reshape(NB)

    grid_spec = pltpu.PrefetchScalarGridSpec(
        num_scalar_prefetch=2,
        grid=(NB,),
        in_specs=[
            pl.BlockSpec((NP, 1), lambda b, be_r, act_r: (0, 0)),
            pl.BlockSpec((NP, 1), lambda b, be_r, act_r: (0, 0)),
            pl.BlockSpec((t, d), lambda b, be_r, act_r: (0, 0)),
            pl.BlockSpec((1, ff2, d), lambda b, be_r, act_r: (be_r[b], 0, 0)),
            pl.BlockSpec((1, d, ff), lambda b, be_r, act_r: (be_r[b], 0, 0)),
        ],
        out_specs=pl.BlockSpec((t, d), lambda b, be_r, act_r: (0, 0)),
    )

    out = pl.pallas_call(
        functools.partial(_moe_kernel, ff=ff),
        grid_spec=grid_spec,
        out_shape=jax.ShapeDtypeStruct((t, d), jnp.float32),
    )(be, act, ids, w, x, gate_up_proj, down_proj)
    return out
